# baseline (speedup 1.0000x reference)
import jax
import jax.numpy as jnp
from jax.experimental import pallas as pl
from jax.experimental.pallas import tpu as pltpu
from jax.experimental.pallas import tpu_sc as plsc

_BM = 512
_WIN = 128


def _tc_encode_body(x_ref, wb_ref, bb_ref, wh_ref, bh_ref, st_ref,
                    ind_ref, snb_ref, snt_scr):
    i = pl.program_id(0)
    kk = st_ref.shape[0]
    bf = jnp.bfloat16

    @pl.when(i == 0)
    def _():
        st = st_ref[...]
        n = jnp.sqrt(jnp.sum(st * st, axis=1, keepdims=True))
        sn = (st / jnp.maximum(n, 1e-12)).astype(bf)
        snb_ref[...] = sn
        snt_scr[...] = sn.T

    h = jnp.dot(x_ref[...], wb_ref[...], preferred_element_type=jnp.float32)
    h = jnp.maximum(h + bb_ref[...], 0.0)
    z = jnp.dot(h.astype(bf), wh_ref[...],
                preferred_element_type=jnp.float32)
    z = z + bh_ref[...]
    zn = z / jnp.maximum(jnp.sqrt(jnp.sum(z * z, axis=1, keepdims=True)),
                         1e-12)
    s = jnp.dot(zn.astype(bf), snt_scr[...],
                preferred_element_type=jnp.float32)
    m = jnp.max(s, axis=1, keepdims=True)
    ids = jax.lax.broadcasted_iota(jnp.int32, s.shape, 1)
    ind = jnp.min(jnp.where(s == m, ids, kk), axis=1)
    ind_ref[0, 0, :] = ind.astype(jnp.int32)


def kernel(x, W_body, b_body, W_head, b_head, abs_states):
    bsz, din = x.shape
    feat = W_body.shape[1]
    d = W_head.shape[1]
    k = abs_states.shape[0]
    bm = _BM
    nb = bsz // bm

    xb = x.astype(jnp.bfloat16)
    wbb = W_body.astype(jnp.bfloat16)
    whb = W_head.astype(jnp.bfloat16)
    bb2 = b_body.reshape(1, feat)
    bh2 = b_head.reshape(1, d)

    ind3, snb = pl.pallas_call(
        _tc_encode_body,
        grid=(nb,),
        in_specs=[
            pl.BlockSpec((bm, din), lambda i: (i, 0)),
            pl.BlockSpec((din, feat), lambda i: (0, 0)),
            pl.BlockSpec((1, feat), lambda i: (0, 0)),
            pl.BlockSpec((feat, d), lambda i: (0, 0)),
            pl.BlockSpec((1, d), lambda i: (0, 0)),
            pl.BlockSpec((k, d), lambda i: (0, 0)),
        ],
        out_specs=[
            pl.BlockSpec((1, 1, bm), lambda i: (i, 0, 0)),
            pl.BlockSpec((k, d), lambda i: (0, 0)),
        ],
        out_shape=[
            jax.ShapeDtypeStruct((nb, 1, bm), jnp.int32),
            jax.ShapeDtypeStruct((k, d), jnp.bfloat16),
        ],
        scratch_shapes=[
            pltpu.VMEM((d, k), jnp.bfloat16),
        ],
    )(xb, wbb, bb2, whb, bh2, abs_states)

    ind = ind3.reshape(1, bsz)

    dp = d // 2
    snp = jax.lax.bitcast_convert_type(snb.reshape(k, dp, 2), jnp.int32)

    vector_mesh = plsc.VectorSubcoreMesh(
        core_axis_name="core", subcore_axis_name="subcore")
    win = _WIN

    @pl.kernel(out_type=jax.ShapeDtypeStruct((bsz, dp), jnp.int32),
               mesh=vector_mesh)
    def _sc_gather(sn_hbm, i_hbm, o_hbm):
        def body(i_vmem, o_vmem):
            pltpu.sync_copy(sn_hbm.at[i_vmem.at[0]], o_vmem)

        pltpu.emit_pipeline(
            body,
            grid=(bsz // win,),
            in_specs=[pl.BlockSpec((1, win), index_map=lambda i: (0, i))],
            out_specs=[pl.BlockSpec((win, dp), index_map=lambda i: (i, 0))],
            core_axis_name=("core", "subcore"),
            dimension_semantics=(pltpu.PARALLEL,),
        )(i_hbm, o_hbm)

    outp = _sc_gather(snp, ind)
    outb = jax.lax.bitcast_convert_type(outp, jnp.bfloat16).reshape(bsz, d)
    return outb.astype(jnp.float32)


# speedup vs baseline: 1.5964x; 1.5964x over previous
import jax
import jax.numpy as jnp
from jax.experimental import pallas as pl
from jax.experimental.pallas import tpu as pltpu
from jax.experimental.pallas import tpu_sc as plsc

_BM = 1024
_WIN = 128


def _tc_encode_body(x_ref, wb_ref, bb_ref, wh_ref, bh_ref, st_ref,
                    ind_ref, sn_ref, wb_scr, wh_scr, snt_scr, sn_scr):
    i = pl.program_id(0)
    kk = st_ref.shape[0]
    bf = jnp.bfloat16

    @pl.when(i == 0)
    def _():
        st = st_ref[...]
        n = jnp.sqrt(jnp.sum(st * st, axis=1, keepdims=True))
        sn = st / jnp.maximum(n, 1e-12)
        sn_scr[...] = sn
        sn_ref[...] = sn
        snt_scr[...] = sn.astype(bf).T
        wb_scr[...] = wb_ref[...].astype(bf)
        wh_scr[...] = wh_ref[...].astype(bf)

    h = jnp.dot(x_ref[...].astype(bf), wb_scr[...],
                preferred_element_type=jnp.float32)
    h = jnp.maximum(h + bb_ref[...], 0.0)
    z = jnp.dot(h.astype(bf), wh_scr[...],
                preferred_element_type=jnp.float32)
    z = z + bh_ref[...]
    zn = z / jnp.maximum(jnp.sqrt(jnp.sum(z * z, axis=1, keepdims=True)),
                         1e-12)
    s = jnp.dot(zn.astype(bf), snt_scr[...],
                preferred_element_type=jnp.float32)
    ind = jnp.argmax(s, axis=1)
    ind_ref[0, 0, :] = ind.astype(jnp.int32)


def kernel(x, W_body, b_body, W_head, b_head, abs_states):
    bsz, din = x.shape
    feat = W_body.shape[1]
    d = W_head.shape[1]
    k = abs_states.shape[0]
    bm = _BM
    nb = bsz // bm

    bb2 = b_body.reshape(1, feat)
    bh2 = b_head.reshape(1, d)

    ind3, sn = pl.pallas_call(
        _tc_encode_body,
        grid=(nb,),
        in_specs=[
            pl.BlockSpec((bm, din), lambda i: (i, 0)),
            pl.BlockSpec((din, feat), lambda i: (0, 0)),
            pl.BlockSpec((1, feat), lambda i: (0, 0)),
            pl.BlockSpec((feat, d), lambda i: (0, 0)),
            pl.BlockSpec((1, d), lambda i: (0, 0)),
            pl.BlockSpec((k, d), lambda i: (0, 0)),
        ],
        out_specs=[
            pl.BlockSpec((1, 1, bm), lambda i: (i, 0, 0)),
            pl.BlockSpec((k, d), lambda i: (0, 0)),
        ],
        out_shape=[
            jax.ShapeDtypeStruct((nb, 1, bm), jnp.int32),
            jax.ShapeDtypeStruct((k, d), jnp.float32),
        ],
        scratch_shapes=[
            pltpu.VMEM((din, feat), jnp.bfloat16),
            pltpu.VMEM((feat, d), jnp.bfloat16),
            pltpu.VMEM((d, k), jnp.bfloat16),
            pltpu.VMEM((k, d), jnp.float32),
        ],
    )(x, W_body, bb2, W_head, bh2, abs_states)

    ind = ind3.reshape(1, bsz)

    vector_mesh = plsc.VectorSubcoreMesh(
        core_axis_name="core", subcore_axis_name="subcore")
    win = _WIN

    @pl.kernel(out_type=jax.ShapeDtypeStruct((bsz, d), jnp.float32),
               mesh=vector_mesh)
    def _sc_gather(sn_hbm, i_hbm, o_hbm):
        def body(i_vmem, o_vmem):
            pltpu.sync_copy(sn_hbm.at[i_vmem.at[0]], o_vmem)

        pltpu.emit_pipeline(
            body,
            grid=(bsz // win,),
            in_specs=[pl.BlockSpec((1, win), index_map=lambda i: (0, i))],
            out_specs=[pl.BlockSpec((win, d), index_map=lambda i: (i, 0))],
            core_axis_name=("core", "subcore"),
            dimension_semantics=(pltpu.PARALLEL,),
        )(i_hbm, o_hbm)

    return _sc_gather(sn, ind)
